# baseline (speedup 1.0000x reference)
import functools

import jax
import jax.numpy as jnp
from jax import lax
from jax.experimental import pallas as pl
from jax.experimental.pallas import tpu as pltpu
from jax.experimental.pallas import tpu_sc as plsc

UNITS_X = 512
UNITS_Y = 512
N_CLASSES = 64
BATCH = 16384

_K_TC = 12288
_K_SC = BATCH - _K_TC


_ROWS_PER_BLOCK = 64


def _reduce_body(cc_ref, out_ref):
    cc = cc_ref[...]
    rev_i = lax.broadcasted_iota(jnp.int32, (1, N_CLASSES, 1), 1)
    rev = (float(N_CLASSES - 1) - rev_i.astype(jnp.float32)) * (
        1.0 / N_CLASSES)
    key = cc + rev
    out_ref[...] = jnp.max(key, axis=1)


def _decode_permute_body(m_ref, x0_ref, p32_ref, pbf_ref):
    ki = (m_ref[...] * float(N_CLASSES)).astype(jnp.int32)
    label = (float(N_CLASSES - 1) - (ki & (N_CLASSES - 1)).astype(jnp.float32))
    table = jnp.where(ki < N_CLASSES, -1.0, label)
    x0v = x0_ref[...].reshape(1, UNITS_X)
    iota = lax.broadcasted_iota(jnp.int32, (UNITS_X, 1), 0)
    oht = (x0v == iota).astype(jnp.bfloat16)
    p = lax.dot_general(oht, table.astype(jnp.bfloat16),
                        (((0,), (0,)), ((), ())),
                        preferred_element_type=jnp.float32)
    p32_ref[...] = p
    pbf_ref[...] = p.astype(jnp.bfloat16)


_BM = 512


def _mm_gather_body(x1_ref, pbf_ref, out_ref):
    x1v = x1_ref[...].reshape(1, _BM)
    iota = lax.broadcasted_iota(jnp.int32, (UNITS_X, 1), 0)
    oht = (x1v == iota).astype(jnp.bfloat16)
    out_ref[...] = lax.dot_general(oht, pbf_ref[...],
                                   (((0,), (0,)), ((), ())),
                                   preferred_element_type=jnp.float32)


def _mm_gather(x1_rows, p_bf16):
    return pl.pallas_call(
        _mm_gather_body,
        grid=(_K_TC // _BM,),
        in_specs=[
            pl.BlockSpec((1, 1, _BM), lambda i: (i, 0, 0)),
            pl.BlockSpec((UNITS_X, UNITS_Y), lambda i: (0, 0)),
        ],
        out_specs=pl.BlockSpec((_BM, UNITS_Y), lambda i: (i, 0)),
        out_shape=jax.ShapeDtypeStruct((BATCH, UNITS_Y), jnp.float32),
    )(x1_rows, p_bf16)


_NC = 2
_NS = 16
_NW = _NC * _NS
_BPW = _K_SC // _NW
_CH = 32
_NBUF = 4
_NCH = _BPW // _CH


@functools.lru_cache(maxsize=None)
def _build_sc_gather():
    mesh = plsc.VectorSubcoreMesh(core_axis_name="c", subcore_axis_name="s")

    @functools.partial(
        pl.kernel,
        mesh=mesh,
        out_type=(),
        scratch_types=[
            pltpu.VMEM((_BPW,), jnp.int32),
            *[pltpu.VMEM((_CH, UNITS_Y), jnp.float32)
              for _ in range(_NBUF)],
            pltpu.SemaphoreType.DMA,
            pltpu.SemaphoreType.DMA,
            pltpu.SemaphoreType.DMA,
        ],
    )
    def _sc_gather(p_hbm, x1_hbm, out_hbm, x1_v, *rest):
        bufs = rest[:_NBUF]
        _, sem_g, sem_w = rest[_NBUF:]
        wid = lax.axis_index("s") * _NC + lax.axis_index("c")
        base = _K_TC + wid * _BPW

        pltpu.sync_copy(x1_hbm.at[pl.ds(base, _BPW)], x1_v)

        def gather(i):
            return pltpu.async_copy(
                p_hbm.at[x1_v.at[pl.ds(i * _CH, _CH)]],
                bufs[i % _NBUF], sem_g)

        gathers = [None] * _NCH
        writes = [None] * _NCH
        for i in range(min(_NBUF - 1, _NCH)):
            gathers[i] = gather(i)
        for i in range(_NCH):
            n = i + _NBUF - 1
            if n < _NCH:
                if n - _NBUF >= 0:
                    writes[n - _NBUF].wait()
                gathers[n] = gather(n)
            gathers[i].wait()
            writes[i] = pltpu.async_copy(
                bufs[i % _NBUF], out_hbm.at[pl.ds(base + i * _CH, _CH)],
                sem_w)
        for i in range(max(0, _NCH - _NBUF), _NCH):
            writes[i].wait()

    return _sc_gather


def kernel(class_count, x):
    cc_t = jnp.transpose(class_count, (0, 2, 1))
    m = pl.pallas_call(
        _reduce_body,
        grid=(UNITS_X // _ROWS_PER_BLOCK,),
        in_specs=[pl.BlockSpec(
            (_ROWS_PER_BLOCK, N_CLASSES, UNITS_Y), lambda i: (i, 0, 0))],
        out_specs=pl.BlockSpec((_ROWS_PER_BLOCK, UNITS_Y), lambda i: (i, 0)),
        out_shape=jax.ShapeDtypeStruct((UNITS_X, UNITS_Y), jnp.float32),
    )(cc_t)

    x = x.astype(jnp.int32)
    x0_row = x[0, :UNITS_X].reshape(1, 1, UNITS_X)
    x1_rows = x[1].reshape(BATCH // _BM, 1, _BM)

    p32, pbf = pl.pallas_call(
        _decode_permute_body,
        out_shape=(
            jax.ShapeDtypeStruct((UNITS_X, UNITS_Y), jnp.float32),
            jax.ShapeDtypeStruct((UNITS_X, UNITS_Y), jnp.bfloat16),
        ),
    )(m, x0_row)

    out_mm = _mm_gather(x1_rows, pbf)
    out_ref = jax.new_ref(out_mm)
    _build_sc_gather()(p32, x[1], out_ref)
    return out_ref[...]


# speedup vs baseline: 1.0858x; 1.0858x over previous
import functools

import jax
import jax.numpy as jnp
from jax import lax
from jax.experimental import pallas as pl
from jax.experimental.pallas import tpu as pltpu
from jax.experimental.pallas import tpu_sc as plsc

UNITS_X = 512
UNITS_Y = 512
N_CLASSES = 64
BATCH = 16384

_K_TC = 12288
_K_SC = BATCH - _K_TC


_ROWS_PER_BLOCK = 64


def _reduce_body(cc_ref, out_ref):
    cc = cc_ref[...]
    rev_i = lax.broadcasted_iota(jnp.int32, (1, N_CLASSES, 1), 1)
    rev = (float(N_CLASSES - 1) - rev_i.astype(jnp.float32)) * (
        1.0 / N_CLASSES)
    key = cc + rev
    out_ref[...] = jnp.max(key, axis=1)


def _decode_permute_body(m_ref, x0_ref, p32_ref, pbf_ref):
    ki = (m_ref[...] * float(N_CLASSES)).astype(jnp.int32)
    label = (float(N_CLASSES - 1) - (ki & (N_CLASSES - 1)).astype(jnp.float32))
    table = jnp.where(ki < N_CLASSES, -1.0, label)
    x0v = x0_ref[...].reshape(1, UNITS_X)
    iota = lax.broadcasted_iota(jnp.int32, (UNITS_X, 1), 0)
    oht = (x0v == iota).astype(jnp.bfloat16)
    p = lax.dot_general(oht, table.astype(jnp.bfloat16),
                        (((0,), (0,)), ((), ())),
                        preferred_element_type=jnp.float32)
    p32_ref[...] = p
    pbf_ref[...] = p.astype(jnp.bfloat16)


_BM = 1024


def _mm_gather_body(x1_ref, pbf_ref, out_ref):
    x1v = x1_ref[...].reshape(1, _BM)
    iota = lax.broadcasted_iota(jnp.int32, (UNITS_X, 1), 0)
    oht = (x1v == iota).astype(jnp.bfloat16)
    out_ref[...] = lax.dot_general(oht, pbf_ref[...],
                                   (((0,), (0,)), ((), ())),
                                   preferred_element_type=jnp.float32)


def _mm_gather(x1_rows, p_bf16):
    return pl.pallas_call(
        _mm_gather_body,
        grid=(_K_TC // _BM,),
        in_specs=[
            pl.BlockSpec((1, 1, _BM), lambda i: (i, 0, 0)),
            pl.BlockSpec((UNITS_X, UNITS_Y), lambda i: (0, 0)),
        ],
        out_specs=pl.BlockSpec((_BM, UNITS_Y), lambda i: (i, 0)),
        out_shape=jax.ShapeDtypeStruct((BATCH, UNITS_Y), jnp.float32),
    )(x1_rows, p_bf16)


_NC = 2
_NS = 16
_NW = _NC * _NS
_BPW = _K_SC // _NW
_CH = 32
_NBUF = 4
_NCH = _BPW // _CH


@functools.lru_cache(maxsize=None)
def _build_sc_gather():
    mesh = plsc.VectorSubcoreMesh(core_axis_name="c", subcore_axis_name="s")

    @functools.partial(
        pl.kernel,
        mesh=mesh,
        out_type=(),
        scratch_types=[
            pltpu.VMEM((_BPW,), jnp.int32),
            *[pltpu.VMEM((_CH, UNITS_Y), jnp.float32)
              for _ in range(_NBUF)],
            pltpu.SemaphoreType.DMA,
            pltpu.SemaphoreType.DMA,
            pltpu.SemaphoreType.DMA,
        ],
    )
    def _sc_gather(p_hbm, x1_hbm, out_hbm, x1_v, *rest):
        bufs = rest[:_NBUF]
        _, sem_g, sem_w = rest[_NBUF:]
        wid = lax.axis_index("s") * _NC + lax.axis_index("c")
        base = _K_TC + wid * _BPW

        pltpu.sync_copy(x1_hbm.at[pl.ds(base, _BPW)], x1_v)

        def gather(i):
            return pltpu.async_copy(
                p_hbm.at[x1_v.at[pl.ds(i * _CH, _CH)]],
                bufs[i % _NBUF], sem_g)

        gathers = [None] * _NCH
        writes = [None] * _NCH
        for i in range(min(_NBUF - 1, _NCH)):
            gathers[i] = gather(i)
        for i in range(_NCH):
            n = i + _NBUF - 1
            if n < _NCH:
                if n - _NBUF >= 0:
                    writes[n - _NBUF].wait()
                gathers[n] = gather(n)
            gathers[i].wait()
            writes[i] = pltpu.async_copy(
                bufs[i % _NBUF], out_hbm.at[pl.ds(base + i * _CH, _CH)],
                sem_w)
        for i in range(max(0, _NCH - _NBUF), _NCH):
            writes[i].wait()

    return _sc_gather


def kernel(class_count, x):
    cc_t = jnp.transpose(class_count, (0, 2, 1))
    m = pl.pallas_call(
        _reduce_body,
        grid=(UNITS_X // _ROWS_PER_BLOCK,),
        in_specs=[pl.BlockSpec(
            (_ROWS_PER_BLOCK, N_CLASSES, UNITS_Y), lambda i: (i, 0, 0))],
        out_specs=pl.BlockSpec((_ROWS_PER_BLOCK, UNITS_Y), lambda i: (i, 0)),
        out_shape=jax.ShapeDtypeStruct((UNITS_X, UNITS_Y), jnp.float32),
    )(cc_t)

    x = x.astype(jnp.int32)
    x0_row = x[0, :UNITS_X].reshape(1, 1, UNITS_X)
    x1_rows = x[1].reshape(BATCH // _BM, 1, _BM)

    p32, pbf = pl.pallas_call(
        _decode_permute_body,
        out_shape=(
            jax.ShapeDtypeStruct((UNITS_X, UNITS_Y), jnp.float32),
            jax.ShapeDtypeStruct((UNITS_X, UNITS_Y), jnp.bfloat16),
        ),
    )(m, x0_row)

    out_mm = _mm_gather(x1_rows, pbf)
    out_ref = jax.new_ref(out_mm)
    _build_sc_gather()(p32, x[1], out_ref)
    return out_ref[...]
